# baseline (speedup 1.0000x reference)
import functools
from math import sqrt

import jax
import jax.numpy as jnp
from jax import lax
from jax.experimental import pallas as pl
from jax.experimental.pallas import tpu as pltpu
from jax.experimental.pallas import tpu_sc as plsc

D_MODEL = 64
SCALE = sqrt(D_MODEL)

NC = 2
NS = 16
NW = NC * NS
LANES = 16

CHUNK = 128


@functools.lru_cache(maxsize=None)
def _build(B, V):
    assert B % (NW * CHUNK) == 0
    b_per_w = B // NW
    n_chunks = b_per_w // CHUNK
    mesh = plsc.VectorSubcoreMesh(core_axis_name="c", subcore_axis_name="s")

    @functools.partial(
        pl.kernel,
        out_type=jax.ShapeDtypeStruct((B, D_MODEL), jnp.float32),
        mesh=mesh,
        scratch_types=[
            pltpu.VMEM((n_chunks, CHUNK), jnp.int32),
            pltpu.VMEM((CHUNK, D_MODEL), jnp.float32),
            pltpu.SemaphoreType.DMA,
        ],
        compiler_params=pltpu.CompilerParams(use_tc_tiling_on_sc=False),
    )
    def emb_kernel(x_hbm, w_hbm, out_hbm, idx_v, rows_v, gsem):
        wid = lax.axis_index("s") * NC + lax.axis_index("c")
        base = wid * b_per_w
        pltpu.sync_copy(x_hbm.at[wid], idx_v)

        def chunk_body(g, carry):
            pltpu.async_copy(w_hbm.at[idx_v.at[g]], rows_v, gsem).wait()

            def row_body(i, c):
                for j in range(D_MODEL // LANES):
                    sl = pl.ds(j * LANES, LANES)
                    rows_v[i, sl] = rows_v[i, sl] * SCALE
                return c

            lax.fori_loop(0, CHUNK, row_body, 0)
            pltpu.sync_copy(rows_v, out_hbm.at[pl.ds(base + g * CHUNK, CHUNK)])
            return carry

        lax.fori_loop(0, n_chunks, chunk_body, 0)

    return emb_kernel


def kernel(x, W):
    batch, seq = x.shape
    B = batch * seq
    x_flat = x.reshape(NW, B // NW // CHUNK, CHUNK).astype(jnp.int32)
    out = _build(B, W.shape[0])(x_flat, W)
    return out.reshape(batch, seq, D_MODEL)


# speedup vs baseline: 1.2072x; 1.2072x over previous
import functools
from math import sqrt

import jax
import jax.numpy as jnp
from jax import lax
from jax.experimental import pallas as pl
from jax.experimental.pallas import tpu as pltpu
from jax.experimental.pallas import tpu_sc as plsc

D_MODEL = 64
SCALE = sqrt(D_MODEL)

NC = 2
NS = 16
NW = NC * NS
LANES = 16

CHUNK = 128
NBUF = 4


@functools.lru_cache(maxsize=None)
def _build(B, V):
    assert B % (NW * CHUNK * NBUF) == 0
    b_per_w = B // NW
    n_chunks = b_per_w // CHUNK
    n_outer = n_chunks // NBUF
    mesh = plsc.VectorSubcoreMesh(core_axis_name="c", subcore_axis_name="s")

    @functools.partial(
        pl.kernel,
        out_type=jax.ShapeDtypeStruct((B, D_MODEL), jnp.float32),
        mesh=mesh,
        scratch_types=[
            pltpu.VMEM((n_chunks, CHUNK), jnp.int32),
            pltpu.VMEM((NBUF, CHUNK, D_MODEL), jnp.float32),
            pltpu.VMEM((NBUF, CHUNK, D_MODEL), jnp.float32),
            pltpu.SemaphoreType.DMA,
            pltpu.SemaphoreType.DMA,
        ],
        compiler_params=pltpu.CompilerParams(use_tc_tiling_on_sc=False),
    )
    def emb_kernel(x_hbm, w_hbm, out_hbm, idx_v, in_v, out_v, gsem, osem):
        wid = lax.axis_index("s") * NC + lax.axis_index("c")
        base = wid * b_per_w
        pltpu.sync_copy(x_hbm.at[wid], idx_v)

        for b in range(NBUF):
            pltpu.async_copy(w_hbm.at[idx_v.at[b]], in_v.at[b], gsem)

        def outer(t, carry):
            for b in range(NBUF):
                g = t * NBUF + b
                pltpu.make_async_copy(
                    w_hbm.at[idx_v.at[b]], in_v.at[b], gsem
                ).wait()

                @pl.when(t > 0)
                def _wait_out():
                    pltpu.make_async_copy(
                        out_v.at[b], out_hbm.at[pl.ds(base, CHUNK)], osem
                    ).wait()

                @plsc.parallel_loop(0, CHUNK, step=1, unroll=8)
                def _scale(i):
                    for j in range(D_MODEL // LANES):
                        sl = pl.ds(j * LANES, LANES)
                        out_v[b, i, sl] = in_v[b, i, sl] * SCALE

                pltpu.async_copy(
                    out_v.at[b], out_hbm.at[pl.ds(base + g * CHUNK, CHUNK)], osem
                )

                @pl.when(g + NBUF < n_chunks)
                def _next_gather():
                    pltpu.async_copy(
                        w_hbm.at[idx_v.at[g + NBUF]], in_v.at[b], gsem
                    )

            return carry

        lax.fori_loop(0, n_outer, outer, 0)

        for b in range(NBUF):
            pltpu.make_async_copy(
                out_v.at[b], out_hbm.at[pl.ds(base, CHUNK)], osem
            ).wait()

    return emb_kernel


def kernel(x, W):
    batch, seq = x.shape
    B = batch * seq
    x_flat = x.reshape(NW, B // NW // CHUNK, CHUNK).astype(jnp.int32)
    out = _build(B, W.shape[0])(x_flat, W)
    return out.reshape(batch, seq, D_MODEL)


# speedup vs baseline: 1.2103x; 1.0026x over previous
import functools
from math import sqrt

import jax
import jax.numpy as jnp
from jax import lax
from jax.experimental import pallas as pl
from jax.experimental.pallas import tpu as pltpu
from jax.experimental.pallas import tpu_sc as plsc

D_MODEL = 64
SCALE = sqrt(D_MODEL)

NC = 2
NS = 16
NW = NC * NS
LANES = 16

CHUNK = 128
NBUF = 4


@functools.lru_cache(maxsize=None)
def _build(B, V):
    assert B % (NW * CHUNK * NBUF) == 0
    b_per_w = B // NW
    n_chunks = b_per_w // CHUNK
    n_outer = n_chunks // NBUF
    mesh = plsc.VectorSubcoreMesh(core_axis_name="c", subcore_axis_name="s")

    @functools.partial(
        pl.kernel,
        out_type=jax.ShapeDtypeStruct((B, D_MODEL), jnp.float32),
        mesh=mesh,
        scratch_types=[
            pltpu.VMEM((n_chunks, CHUNK), jnp.int32),
            pltpu.VMEM((NBUF, CHUNK, D_MODEL), jnp.float32),
            pltpu.VMEM((NBUF, CHUNK, D_MODEL), jnp.float32),
            pltpu.SemaphoreType.DMA,
            pltpu.SemaphoreType.DMA,
        ],
        compiler_params=pltpu.CompilerParams(
            use_tc_tiling_on_sc=False,
            skip_device_barrier=True,
        ),
    )
    def emb_kernel(x_hbm, w_hbm, out_hbm, idx_v, in_v, out_v, gsem, osem):
        wid = lax.axis_index("s") * NC + lax.axis_index("c")
        base = wid * b_per_w
        pltpu.sync_copy(x_hbm.at[wid], idx_v)

        for b in range(NBUF):
            pltpu.async_copy(w_hbm.at[idx_v.at[b]], in_v.at[b], gsem)

        def outer(t, carry):
            for b in range(NBUF):
                g = t * NBUF + b
                pltpu.make_async_copy(
                    w_hbm.at[idx_v.at[b]], in_v.at[b], gsem
                ).wait()

                @pl.when(t > 0)
                def _wait_out():
                    pltpu.make_async_copy(
                        out_v.at[b], out_hbm.at[pl.ds(base, CHUNK)], osem
                    ).wait()

                @plsc.parallel_loop(0, CHUNK, step=1, unroll=8)
                def _scale(i):
                    for j in range(D_MODEL // LANES):
                        sl = pl.ds(j * LANES, LANES)
                        out_v[b, i, sl] = in_v[b, i, sl] * SCALE

                pltpu.async_copy(
                    out_v.at[b], out_hbm.at[pl.ds(base + g * CHUNK, CHUNK)], osem
                )

                @pl.when(g + NBUF < n_chunks)
                def _next_gather():
                    pltpu.async_copy(
                        w_hbm.at[idx_v.at[g + NBUF]], in_v.at[b], gsem
                    )

            return carry

        lax.fori_loop(0, n_outer, outer, 0)

        for b in range(NBUF):
            pltpu.make_async_copy(
                out_v.at[b], out_hbm.at[pl.ds(base, CHUNK)], osem
            ).wait()

    return emb_kernel


def kernel(x, W):
    batch, seq = x.shape
    B = batch * seq
    x_flat = x.reshape(NW, B // NW // CHUNK, CHUNK).astype(jnp.int32)
    out = _build(B, W.shape[0])(x_flat, W)
    return out.reshape(batch, seq, D_MODEL)
